# baseline (speedup 1.0000x reference)
import functools

import jax
import jax.numpy as jnp
from jax import lax
from jax.experimental import pallas as pl
from jax.experimental.pallas import tpu as pltpu
from jax.experimental.pallas import tpu_sc as plsc

D = 64
NC = 2
NS = 16
NW = NC * NS
CH = 128


@functools.lru_cache(maxsize=None)
def _make(B):
    BPW = B // NW
    NCH = BPW // CH
    mesh = plsc.VectorSubcoreMesh(core_axis_name="c", subcore_axis_name="s")

    @functools.partial(
        pl.kernel,
        mesh=mesh,
        out_type=jax.ShapeDtypeStruct((B, D), jnp.float32),
        scratch_types=[
            pltpu.VMEM((NCH, CH), jnp.int32),
            pltpu.VMEM((CH, D), jnp.float32),
            pltpu.SemaphoreType.DMA,
        ],
        compiler_params=pltpu.CompilerParams(use_tc_tiling_on_sc=False),
    )
    def k(table_hbm, idx_hbm, out_hbm, idx_v, rows_v, sem):
        wid = lax.axis_index("s") * NC + lax.axis_index("c")
        base = wid * BPW
        pltpu.sync_copy(idx_hbm.at[wid], idx_v)

        def chunk(c, carry):
            pltpu.async_copy(table_hbm.at[idx_v.at[c]], rows_v, sem).wait()
            pltpu.sync_copy(rows_v, out_hbm.at[pl.ds(base + c * CH, CH)])
            return carry

        lax.fori_loop(0, NCH, chunk, 0)

    return k


def kernel(in_tensor, table):
    B = in_tensor.shape[0] * in_tensor.shape[1]
    idx = in_tensor.reshape(NW, B // (NW * CH), CH)
    out = _make(B)(table, idx)
    return out.reshape(in_tensor.shape + (D,))


# speedup vs baseline: 1.0459x; 1.0459x over previous
import functools

import jax
import jax.numpy as jnp
from jax import lax
from jax.experimental import pallas as pl
from jax.experimental.pallas import tpu as pltpu
from jax.experimental.pallas import tpu_sc as plsc

D = 64
NC = 2
NS = 16
NW = NC * NS
CH = 640


@functools.lru_cache(maxsize=None)
def _make(B):
    BPW = B // NW
    NCH = BPW // CH
    mesh = plsc.VectorSubcoreMesh(core_axis_name="c", subcore_axis_name="s")

    @functools.partial(
        pl.kernel,
        mesh=mesh,
        out_type=jax.ShapeDtypeStruct((B, D), jnp.float32),
        scratch_types=[
            pltpu.VMEM((NCH, CH), jnp.int32),
            pltpu.VMEM((CH, D), jnp.float32),
            pltpu.VMEM((CH, D), jnp.float32),
            pltpu.SemaphoreType.DMA,
            pltpu.SemaphoreType.DMA,
            pltpu.SemaphoreType.DMA,
            pltpu.SemaphoreType.DMA,
        ],
        compiler_params=pltpu.CompilerParams(use_tc_tiling_on_sc=False),
    )
    def k(table_hbm, idx_hbm, out_hbm, idx_v, buf0, buf1, gs0, gs1, ws0, ws1):
        wid = lax.axis_index("s") * NC + lax.axis_index("c")
        base = wid * BPW
        pltpu.sync_copy(idx_hbm.at[wid], idx_v)

        bufs = (buf0, buf1)
        gsems = (gs0, gs1)
        wsems = (ws0, ws1)

        def fire(c, p):
            return pltpu.async_copy(table_hbm.at[idx_v.at[c]], bufs[p], gsems[p])

        pend_g = [fire(0, 0), None]
        pend_w = [None, None]
        for c in range(NCH):
            p = c % 2
            if c + 1 < NCH:
                q = p ^ 1
                if pend_w[q] is not None:
                    pend_w[q].wait()
                pend_g[q] = fire(c + 1, q)
            pend_g[p].wait()
            pend_w[p] = pltpu.async_copy(
                bufs[p], out_hbm.at[pl.ds(base + c * CH, CH)], wsems[p])
        for w in pend_w:
            if w is not None:
                w.wait()

    return k


def kernel(in_tensor, table):
    B = in_tensor.shape[0] * in_tensor.shape[1]
    idx = in_tensor.reshape(NW, B // (NW * CH), CH)
    out = _make(B)(table, idx)
    return out.reshape(in_tensor.shape + (D,))
